# baseline (speedup 1.0000x reference)
import functools
import math

import jax
import jax.numpy as jnp
from jax import lax
from jax.experimental import pallas as pl
from jax.experimental.pallas import tpu as pltpu
from jax.experimental.pallas import tpu_sc as plsc

_VOCAB = 100000
_DIM = 128
_MAX_LEN = 512
_BATCH = 1024
_SCALE = math.sqrt(float(_DIM))

_NC = 2
_NS = 16
_L = 16
_NW = _NC * _NS
_TOTAL = _BATCH * _MAX_LEN
_ROWS_PER_W = _TOTAL // _NW
_CHUNK = 64
_NCHUNK = _ROWS_PER_W // _CHUNK
_NBUF = 4


@functools.partial(
    pl.kernel,
    mesh=plsc.VectorSubcoreMesh(core_axis_name="c", subcore_axis_name="s"),
    out_type=jax.ShapeDtypeStruct((_TOTAL, _DIM), jnp.float32),
    scratch_types=[
        pltpu.VMEM((_MAX_LEN, _DIM), jnp.float32),
        pltpu.VMEM((_ROWS_PER_W,), jnp.int32),
    ] + [pltpu.VMEM((_CHUNK, _DIM), jnp.float32)] * _NBUF
      + [pltpu.SemaphoreType.DMA] * (2 * _NBUF),
)
def _embed(idx_hbm, table_hbm, pos_hbm, out_hbm, pos_v, idx_v, *bufs_sems):
    rows = bufs_sems[:_NBUF]
    gs = bufs_sems[_NBUF:2 * _NBUF]
    os_ = bufs_sems[2 * _NBUF:]
    wid = lax.axis_index("s") * _NC + lax.axis_index("c")
    base = wid * _ROWS_PER_W
    pltpu.sync_copy(idx_hbm.at[pl.ds(base, _ROWS_PER_W)], idx_v)
    pltpu.sync_copy(pos_hbm, pos_v)

    def gather(c, b):
        return pltpu.make_async_copy(
            table_hbm.at[idx_v.at[pl.ds(c * _CHUNK, _CHUNK)]], rows[b], gs[b])

    def out_copy(c, b):
        return pltpu.make_async_copy(
            rows[b], out_hbm.at[pl.ds(base + c * _CHUNK, _CHUNK)], os_[b])

    def compute(c, b):
        p0 = lax.rem(c * _CHUNK, _MAX_LEN)
        buf = rows[b]

        @plsc.parallel_loop(0, _CHUNK, 1, unroll=8)
        def _(r):
            for j in range(_DIM // _L):
                sl = pl.ds(j * _L, _L)
                buf[r, sl] = buf[r, sl] * _SCALE + pos_v[p0 + r, sl]

    gather(0, 0).start()
    gather(1, 1).start()

    def group_body(g, carry):
        for b in range(_NBUF):
            c = _NBUF * g + b
            gather(c, b).wait()
            compute(c, b)
            out_copy(c, b).start()

            @pl.when(c + 2 < _NCHUNK)
            def _():
                b2 = (b + 2) % _NBUF

                @pl.when(c >= 2)
                def _():
                    out_copy(c - 2, b2).wait()

                gather(c + 2, b2).start()

        return carry

    lax.fori_loop(0, _NCHUNK // _NBUF, group_body, 0)
    for k in range(_NBUF):
        c = _NCHUNK - _NBUF + k
        out_copy(c, c % _NBUF).wait()


def kernel(inputs, table, pos_encoding):
    inputs = inputs[:, :_MAX_LEN]
    idx = inputs.reshape(-1)
    out = _embed(idx, table, pos_encoding)
    return out.reshape(inputs.shape[0], inputs.shape[1], _DIM)


# speedup vs baseline: 1.0053x; 1.0053x over previous
import functools
import math

import jax
import jax.numpy as jnp
from jax import lax
from jax.experimental import pallas as pl
from jax.experimental.pallas import tpu as pltpu
from jax.experimental.pallas import tpu_sc as plsc

_VOCAB = 100000
_DIM = 128
_MAX_LEN = 512
_BATCH = 1024
_SCALE = math.sqrt(float(_DIM))

_NC = 2
_NS = 16
_L = 16
_NW = _NC * _NS
_TOTAL = _BATCH * _MAX_LEN
_ROWS_PER_W = _TOTAL // _NW
_CHUNK = 64
_NCHUNK = _ROWS_PER_W // _CHUNK
_NBUF = 4


@functools.partial(
    pl.kernel,
    mesh=plsc.VectorSubcoreMesh(core_axis_name="c", subcore_axis_name="s"),
    out_type=jax.ShapeDtypeStruct((_TOTAL, _DIM), jnp.float32),
    scratch_types=[
        pltpu.VMEM((_MAX_LEN, _DIM), jnp.float32),
        pltpu.VMEM((_ROWS_PER_W,), jnp.int32),
    ] + [pltpu.VMEM((_CHUNK, _DIM), jnp.float32)] * _NBUF
      + [pltpu.SemaphoreType.DMA] * (2 * _NBUF),
)
def _embed(idx_hbm, table_hbm, pos_hbm, out_hbm, pos_v, idx_v, *bufs_sems):
    rows = bufs_sems[:_NBUF]
    gs = bufs_sems[_NBUF:2 * _NBUF]
    os_ = bufs_sems[2 * _NBUF:]
    wid = lax.axis_index("s") * _NC + lax.axis_index("c")
    base = wid * _ROWS_PER_W
    pltpu.sync_copy(idx_hbm.at[pl.ds(base, _ROWS_PER_W)], idx_v)
    pltpu.sync_copy(pos_hbm, pos_v)

    def gather(c, b):
        return pltpu.make_async_copy(
            table_hbm.at[idx_v.at[pl.ds(c * _CHUNK, _CHUNK)]], rows[b], gs[b])

    def out_copy(c, b):
        return pltpu.make_async_copy(
            rows[b], out_hbm.at[pl.ds(base + c * _CHUNK, _CHUNK)], os_[b])

    def compute(c, b):
        p0 = lax.rem(c * _CHUNK, _MAX_LEN)
        buf = rows[b]

        @plsc.parallel_loop(0, _CHUNK, 1, unroll=2)
        def _(r):
            for j in range(_DIM // _L):
                sl = pl.ds(j * _L, _L)
                buf[r, sl] = buf[r, sl] * _SCALE + pos_v[p0 + r, sl]

    gather(0, 0).start()
    gather(1, 1).start()

    def group_body(g, carry):
        for b in range(_NBUF):
            c = _NBUF * g + b
            gather(c, b).wait()
            compute(c, b)
            out_copy(c, b).start()

            @pl.when(c + 2 < _NCHUNK)
            def _():
                b2 = (b + 2) % _NBUF

                @pl.when(c >= 2)
                def _():
                    out_copy(c - 2, b2).wait()

                gather(c + 2, b2).start()

        return carry

    lax.fori_loop(0, _NCHUNK // _NBUF, group_body, 0)
    for k in range(_NBUF):
        c = _NCHUNK - _NBUF + k
        out_copy(c, c % _NBUF).wait()


def kernel(inputs, table, pos_encoding):
    inputs = inputs[:, :_MAX_LEN]
    idx = inputs.reshape(-1)
    out = _embed(idx, table, pos_encoding)
    return out.reshape(inputs.shape[0], inputs.shape[1], _DIM)


# speedup vs baseline: 1.0112x; 1.0059x over previous
import functools
import math

import jax
import jax.numpy as jnp
from jax import lax
from jax.experimental import pallas as pl
from jax.experimental.pallas import tpu as pltpu
from jax.experimental.pallas import tpu_sc as plsc

_VOCAB = 100000
_DIM = 128
_MAX_LEN = 512
_BATCH = 1024
_SCALE = math.sqrt(float(_DIM))

_NC = 2
_NS = 16
_L = 16
_NW = _NC * _NS
_TOTAL = _BATCH * _MAX_LEN
_ROWS_PER_W = _TOTAL // _NW
_CHUNK = 64
_NCHUNK = _ROWS_PER_W // _CHUNK
_NBUF = 4


@functools.partial(
    pl.kernel,
    mesh=plsc.VectorSubcoreMesh(core_axis_name="c", subcore_axis_name="s"),
    out_type=jax.ShapeDtypeStruct((_TOTAL, _DIM), jnp.float32),
    scratch_types=[
        pltpu.VMEM((_MAX_LEN, _DIM), jnp.float32),
        pltpu.VMEM((_ROWS_PER_W,), jnp.int32),
    ] + [pltpu.VMEM((_CHUNK, _DIM), jnp.float32)] * _NBUF
      + [pltpu.SemaphoreType.DMA] * (2 * _NBUF),
)
def _embed(idx_hbm, table_hbm, pos_hbm, out_hbm, pos_v, idx_v, *bufs_sems):
    rows = bufs_sems[:_NBUF]
    gs = bufs_sems[_NBUF:2 * _NBUF]
    os_ = bufs_sems[2 * _NBUF:]
    wid = lax.axis_index("s") * _NC + lax.axis_index("c")
    base = wid * _ROWS_PER_W
    idx_load = pltpu.make_async_copy(
        idx_hbm.at[pl.ds(base, _ROWS_PER_W)], idx_v, os_[1])
    pos_load = pltpu.make_async_copy(pos_hbm, pos_v, os_[0])
    idx_load.start()
    pos_load.start()

    def gather(c, b):
        return pltpu.make_async_copy(
            table_hbm.at[idx_v.at[pl.ds(c * _CHUNK, _CHUNK)]], rows[b], gs[b])

    def out_copy(c, b):
        return pltpu.make_async_copy(
            rows[b], out_hbm.at[pl.ds(base + c * _CHUNK, _CHUNK)], os_[b])

    def compute(c, b):
        p0 = lax.rem(c * _CHUNK, _MAX_LEN)
        buf = rows[b]

        @plsc.parallel_loop(0, _CHUNK, 1, unroll=4)
        def _(r):
            for j in range(_DIM // _L):
                sl = pl.ds(j * _L, _L)
                buf[r, sl] = buf[r, sl] * _SCALE + pos_v[p0 + r, sl]

    idx_load.wait()
    gather(0, 0).start()
    gather(1, 1).start()
    pos_load.wait()

    def group_body(g, carry):
        for b in range(_NBUF):
            c = _NBUF * g + b
            gather(c, b).wait()
            compute(c, b)
            out_copy(c, b).start()

            @pl.when(c + 2 < _NCHUNK)
            def _():
                b2 = (b + 2) % _NBUF

                @pl.when(c >= 2)
                def _():
                    out_copy(c - 2, b2).wait()

                gather(c + 2, b2).start()

        return carry

    lax.fori_loop(0, _NCHUNK // _NBUF, group_body, 0)
    for k in range(_NBUF):
        c = _NCHUNK - _NBUF + k
        out_copy(c, c % _NBUF).wait()


def kernel(inputs, table, pos_encoding):
    inputs = inputs[:, :_MAX_LEN]
    idx = inputs.reshape(-1)
    out = _embed(idx, table, pos_encoding)
    return out.reshape(inputs.shape[0], inputs.shape[1], _DIM)
